# baseline (speedup 1.0000x reference)
import functools

import jax
import jax.numpy as jnp
from jax import lax
from jax.experimental import pallas as pl
from jax.experimental.pallas import tpu as pltpu
from jax.experimental.pallas import tpu_sc as plsc

N = 10000
E = 320000
G = 16
C = 2

NC = 2
NS = 16
NW = NC * NS
K = 64
NB_E = 160
NBUF = 4
LA = NBUF - 1
EPT = NB_E * K
EPAD = NW * EPT
DUMMY = N
NPAD = 10112
ROWS_PER_TILE = NPAD // NS


@functools.lru_cache(maxsize=None)
def _make_sc_agg(Dc):
    mesh = plsc.VectorSubcoreMesh(core_axis_name="c", subcore_axis_name="s")

    @functools.partial(
        pl.kernel,
        out_type=jax.ShapeDtypeStruct((NC, Dc, NPAD, 128), jnp.float32),
        mesh=mesh,
        scratch_types=[
            pltpu.VMEM((NB_E // 2, K), jnp.int32),
            pltpu.VMEM((NBUF, 2, K), jnp.int32),
            pltpu.VMEM((K, 128), jnp.float32),
            pltpu.VMEM((K, 128), jnp.float32),
            pltpu.VMEM((K, 128), jnp.float32),
            pltpu.VMEM((K, 128), jnp.float32),
            pltpu.VMEM_SHARED((NPAD, 128), jnp.float32),
            pltpu.SemaphoreType.DMA,
            pltpu.SemaphoreType.DMA,
            pltpu.SemaphoreType.DMA,
            pltpu.SemaphoreType.DMA,
        ],
    )
    def agg(h_hbm, pk_hbm, p_hbm, pk_t, sd_idx, rows0, rows1, rows2, rows3,
            agg_sh, sem0, sem1, sem2, sem3):
        cid = lax.axis_index("c")
        sid = lax.axis_index("s")
        r0 = sid * ROWS_PER_TILE
        rows = (rows0, rows1, rows2, rows3)
        sems = (sem0, sem1, sem2, sem3)

        HALF_E = NB_E // 2

        def _unpack(j, slot):
            for q in range(K // 16):
                p = pk_t[j % HALF_E, pl.ds(q * 16, 16)]
                sd_idx[slot, 0, pl.ds(q * 16, 16)] = p & 0xFFFF
                sd_idx[slot, 1, pl.ds(q * 16, 16)] = p >> 16

        for c in range(Dc):
            def _z(i, carry):
                rows0[i // 8, pl.ds((i % 8) * 16, 16)] = jnp.zeros(
                    (16,), jnp.float32)
                return carry

            lax.fori_loop(0, K * 8, _z, 0)
            for z in range(ROWS_PER_TILE // K):
                pltpu.sync_copy(rows0, agg_sh.at[pl.ds(r0 + z * K, K)])
            rem = ROWS_PER_TILE % K
            if rem:
                pltpu.sync_copy(
                    rows0.at[pl.ds(0, rem)],
                    agg_sh.at[pl.ds(r0 + ROWS_PER_TILE - rem, rem)])
            plsc.subcore_barrier()

            hc = h_hbm.at[c]

            pltpu.sync_copy(pk_hbm.at[cid, sid, pl.ds(0, HALF_E)], pk_t)

            for t in range(LA):
                _unpack(t, t)
                pltpu.async_copy(hc.at[sd_idx.at[t, 0]], rows[t], sems[t])

            def _quad(m, carry):
                for k in range(NBUF):
                    b = NBUF * m + k
                    slot = (k + LA) % NBUF

                    @pl.when(b + LA < NB_E)
                    def _():
                        @pl.when(b + LA == HALF_E)
                        def _():
                            pltpu.sync_copy(
                                pk_hbm.at[cid, sid, pl.ds(HALF_E, HALF_E)],
                                pk_t)

                        _unpack(b + LA, slot)
                        pltpu.async_copy(
                            hc.at[sd_idx.at[slot, 0]], rows[slot], sems[slot])

                    pltpu.make_async_copy(
                        hc.at[sd_idx.at[k, 0]], rows[k], sems[k]).wait()
                    pltpu.sync_copy(
                        rows[k], agg_sh.at[sd_idx.at[k, 1]], add=True)
                return carry

            lax.fori_loop(0, NB_E // NBUF, _quad, 0)
            plsc.subcore_barrier()

            pltpu.sync_copy(
                agg_sh.at[pl.ds(r0, ROWS_PER_TILE)],
                p_hbm.at[cid, c, pl.ds(r0, ROWS_PER_TILE)],
            )

    return agg


CPC = 2
NB2 = 320
SEG = 80


@functools.lru_cache(maxsize=None)
def _make_sc_agg_owned(cp):
    mesh = plsc.VectorSubcoreMesh(core_axis_name="c", subcore_axis_name="s")

    @functools.partial(
        pl.kernel,
        out_type=jax.ShapeDtypeStruct((2, NPAD, 128), jnp.float32),
        mesh=mesh,
        scratch_types=[
            pltpu.VMEM((SEG, K), jnp.int32),
            pltpu.VMEM((NBUF, 2, K), jnp.int32),
            pltpu.VMEM((K, 128), jnp.float32),
            pltpu.VMEM((K, 128), jnp.float32),
            pltpu.VMEM((K, 128), jnp.float32),
            pltpu.VMEM((K, 128), jnp.float32),
            pltpu.VMEM_SHARED((NPAD, 128), jnp.float32),
            pltpu.SemaphoreType.DMA,
            pltpu.SemaphoreType.DMA,
            pltpu.SemaphoreType.DMA,
            pltpu.SemaphoreType.DMA,
        ],
    )
    def agg(h_hbm, pk_hbm, p_hbm, pk_t, sd_idx, rows0, rows1, rows2, rows3,
            agg_sh, sem0, sem1, sem2, sem3):
        cid = lax.axis_index("c")
        sid = lax.axis_index("s")
        r0 = sid * ROWS_PER_TILE
        rows = (rows0, rows1, rows2, rows3)
        sems = (sem0, sem1, sem2, sem3)

        if True:
            cbase = (2 * cp + cid) * N

            def _unpack(j, slot):
                for q in range(K // 16):
                    p = pk_t[j % SEG, pl.ds(q * 16, 16)]
                    sd_idx[slot, 0, pl.ds(q * 16, 16)] = (p & 0xFFFF) + cbase
                    sd_idx[slot, 1, pl.ds(q * 16, 16)] = p >> 16

            def _z(i, carry):
                rows0[i // 8, pl.ds((i % 8) * 16, 16)] = jnp.zeros(
                    (16,), jnp.float32)
                return carry

            lax.fori_loop(0, K * 8, _z, 0)
            for z in range(ROWS_PER_TILE // K):
                pltpu.sync_copy(rows0, agg_sh.at[pl.ds(r0 + z * K, K)])
            rem = ROWS_PER_TILE % K
            if rem:
                pltpu.sync_copy(
                    rows0.at[pl.ds(0, rem)],
                    agg_sh.at[pl.ds(r0 + ROWS_PER_TILE - rem, rem)])
            plsc.subcore_barrier()

            pltpu.sync_copy(pk_hbm.at[sid, pl.ds(0, SEG)], pk_t)
            for t in range(LA):
                _unpack(t, t)
                pltpu.async_copy(h_hbm.at[sd_idx.at[t, 0]], rows[t], sems[t])

            def _quad(m, carry):
                for k in range(NBUF):
                    b = NBUF * m + k
                    slot = (k + LA) % NBUF

                    @pl.when(b + LA < NB2)
                    def _():
                        jj = b + LA

                        @pl.when(jj % SEG == 0)
                        def _():
                            start = pl.multiple_of(jj, 16)
                            pltpu.sync_copy(
                                pk_hbm.at[sid, pl.ds(start, SEG)], pk_t)

                        _unpack(jj, slot)
                        pltpu.async_copy(
                            h_hbm.at[sd_idx.at[slot, 0]], rows[slot],
                            sems[slot])

                    pltpu.make_async_copy(
                        h_hbm.at[sd_idx.at[k, 0]], rows[k], sems[k]).wait()
                    pltpu.sync_copy(
                        rows[k], agg_sh.at[sd_idx.at[k, 1]], add=True)
                return carry

            lax.fori_loop(0, NB2 // NBUF, _quad, 0)
            plsc.subcore_barrier()

            pltpu.sync_copy(
                agg_sh.at[pl.ds(r0, ROWS_PER_TILE)],
                p_hbm.at[cid, pl.ds(r0, ROWS_PER_TILE)],
            )

    return agg


R = 1000
NBLK = N // R


def _mlp_p1_body(h_ref, p_ref, w_ref, b_ref, o_ref):
    m = jnp.concatenate([h_ref[d] + p_ref[d] for d in range(2)], axis=-1)
    o_ref[...] = jnp.dot(m, w_ref[...], preferred_element_type=jnp.float32) + b_ref[...]


def _mlp_p2_body(h_ref, p_ref, acc_ref, w1_ref, w2_ref, b2_ref, o_ref):
    m = jnp.concatenate([h_ref[d] + p_ref[d] for d in range(2)], axis=-1)
    t = acc_ref[...] + jnp.dot(m, w1_ref[...], preferred_element_type=jnp.float32)
    m1 = jnp.maximum(t, 0.0)
    m2 = jnp.dot(m1, w2_ref[...], preferred_element_type=jnp.float32) + b2_ref[...]
    m2 = jnp.maximum(m2, 0.0)
    for d in range(4):
        o_ref[d] = m2[:, d * 128:(d + 1) * 128]


def _mlp_split(h_c, pA, pB, W1, b1, W2, b2):
    acc = pl.pallas_call(
        _mlp_p1_body,
        grid=(NBLK,),
        in_specs=[
            pl.BlockSpec((2, R, 128), lambda i: (0, i, 0)),
            pl.BlockSpec((2, R, 128), lambda i: (0, i, 0)),
            pl.BlockSpec((256, 512), lambda i: (0, 0)),
            pl.BlockSpec((1, 512), lambda i: (0, 0)),
        ],
        out_specs=pl.BlockSpec((R, 512), lambda i: (i, 0)),
        out_shape=jax.ShapeDtypeStruct((N, 512), jnp.float32),
    )(h_c, pA, W1[:256], b1.reshape(1, 512))
    return pl.pallas_call(
        _mlp_p2_body,
        grid=(NBLK,),
        in_specs=[
            pl.BlockSpec((2, R, 128), lambda i: (1, i, 0)),
            pl.BlockSpec((2, R, 128), lambda i: (0, i, 0)),
            pl.BlockSpec((R, 512), lambda i: (i, 0)),
            pl.BlockSpec((256, 512), lambda i: (0, 0)),
            pl.BlockSpec((512, 512), lambda i: (0, 0)),
            pl.BlockSpec((1, 512), lambda i: (0, 0)),
        ],
        out_specs=pl.BlockSpec((4, R, 128), lambda i: (0, i, 0)),
        out_shape=jax.ShapeDtypeStruct((4, N, 128), jnp.float32),
    )(h_c, pB, acc, W1[256:], W2, b2.reshape(1, 512))


def _mlp_body(Dcin, h_ref, p_ref, w1_ref, b1_ref, w2_ref, b2_ref, o_ref):
    parts = [h_ref[d] + p_ref[0, d] + p_ref[1, d] for d in range(Dcin)]
    m = parts[0] if Dcin == 1 else jnp.concatenate(parts, axis=-1)
    m = jnp.dot(m, w1_ref[...], preferred_element_type=jnp.float32) + b1_ref[...]
    m = jnp.maximum(m, 0.0)
    m = jnp.dot(m, w2_ref[...], preferred_element_type=jnp.float32) + b2_ref[...]
    m = jnp.maximum(m, 0.0)
    for d in range(4):
        o_ref[d] = m[:, d * 128:(d + 1) * 128]


def _mlp(h_c, p, W1, b1, W2, b2):
    Dcin = h_c.shape[0]
    Din = Dcin * 128
    p_spec = (pl.BlockSpec((NC, Dcin, R, 128), lambda i: (0, 0, i, 0))
              if p.ndim == 4 else
              pl.BlockSpec((Dcin, R, 128), lambda i: (0, i, 0)))
    return pl.pallas_call(
        functools.partial(_mlp_body, Dcin),
        grid=(NBLK,),
        in_specs=[
            pl.BlockSpec((Dcin, R, 128), lambda i: (0, i, 0)),
            p_spec,
            pl.BlockSpec((Din, 512), lambda i: (0, 0)),
            pl.BlockSpec((1, 512), lambda i: (0, 0)),
            pl.BlockSpec((512, 512), lambda i: (0, 0)),
            pl.BlockSpec((1, 512), lambda i: (0, 0)),
        ],
        out_specs=pl.BlockSpec((4, R, 128), lambda i: (0, i, 0)),
        out_shape=jax.ShapeDtypeStruct((4, N, 128), jnp.float32),
    )(h_c, p, W1, b1.reshape(1, 512), W2, b2.reshape(1, 512))


def _pool_body(h_ref, b_ref, wl_ref, bl_ref, o_ref, g_ref, sums, cnts):
    i = pl.program_id(0)

    @pl.when(i == 0)
    def _():
        sums[...] = jnp.zeros_like(sums)
        cnts[...] = jnp.zeros_like(cnts)

    hb = jnp.concatenate([h_ref[d] for d in range(4)], axis=-1)
    b = b_ref[0]
    oh = (lax.broadcasted_iota(jnp.int32, (G, R), 0) == b).astype(jnp.float32)
    sums[...] += jnp.dot(oh, hb, preferred_element_type=jnp.float32)
    cnts[...] += jnp.broadcast_to(jnp.sum(oh, axis=1, keepdims=True), cnts.shape)

    @pl.when(i == NBLK - 1)
    def _():
        cnt = jnp.maximum(cnts[:, :1], 1.0)
        g = sums[...] / cnt
        g_ref[...] = g
        o_ref[...] = jnp.dot(g, wl_ref[...], preferred_element_type=jnp.float32) + bl_ref[...]


def _pool(h_c, batch_r, Wl, bl):
    return pl.pallas_call(
        _pool_body,
        grid=(NBLK,),
        in_specs=[
            pl.BlockSpec((4, R, 128), lambda i: (0, i, 0)),
            pl.BlockSpec((1, 1, R), lambda i: (i, 0, 0)),
            pl.BlockSpec((512, 128), lambda i: (0, 0)),
            pl.BlockSpec((1, 128), lambda i: (0, 0)),
        ],
        out_specs=[
            pl.BlockSpec((G, 128), lambda i: (0, 0)),
            pl.BlockSpec((G, 512), lambda i: (0, 0)),
        ],
        out_shape=[
            jax.ShapeDtypeStruct((G, 128), jnp.float32),
            jax.ShapeDtypeStruct((G, 512), jnp.float32),
        ],
        scratch_shapes=[
            pltpu.VMEM((G, 512), jnp.float32),
            pltpu.VMEM((G, 128), jnp.float32),
        ],
    )(h_c, batch_r, Wl, bl)


@jax.jit
def kernel(x, edge_index, batch,
           W1_0, b1_0, W2_0, b2_0,
           W1_1, b1_1, W2_1, b2_1,
           W1_2, b1_2, W2_2, b2_2,
           W_lin, b_lin):
    pad_i = lax.iota(jnp.int32, EPAD - E)
    src_p = jnp.concatenate([edge_index[0], pad_i % N])
    dst_p = jnp.concatenate([edge_index[1], DUMMY + pad_i % (NPAD - DUMMY)])
    packed = src_p | (dst_p << 16)
    pk1 = packed.reshape(NC, NS, NB_E, K)
    pk2 = packed.reshape(NS, NB2, K)

    h = x.reshape(1, N, 128)
    layer_params = [(W1_0, b1_0, W2_0, b2_0),
                    (W1_1, b1_1, W2_1, b2_1),
                    (W1_2, b1_2, W2_2, b2_2)]
    for li, (W1, b1, W2, b2) in enumerate(layer_params):
        if h.shape[0] == 1:
            p = _make_sc_agg(1)(h, pk1)
            h = _mlp(h, p, W1, b1, W2, b2)
        else:
            h2d = h.reshape(4 * N, 128)
            pA = _make_sc_agg_owned(0)(h2d, pk2)
            pB = _make_sc_agg_owned(1)(h2d, pk2)
            h = _mlp_split(h, pA, pB, W1, b1, W2, b2)

    batch_r = batch.reshape(NBLK, 1, R)
    Wl = jnp.pad(W_lin, ((0, 0), (0, 128 - C)))
    bl = jnp.pad(b_lin, (0, 128 - C)).reshape(1, 128)
    out_p, g = _pool(h, batch_r, Wl, bl)
    return (out_p[:, :C], g)


# speedup vs baseline: 1.0140x; 1.0140x over previous
import functools

import jax
import jax.numpy as jnp
from jax import lax
from jax.experimental import pallas as pl
from jax.experimental.pallas import tpu as pltpu
from jax.experimental.pallas import tpu_sc as plsc

N = 10000
E = 320000
G = 16
C = 2

NC = 2
NS = 16
NW = NC * NS
K = 64
NB_E = 160
NBUF = 4
LA = NBUF - 1
EPT = NB_E * K
EPAD = NW * EPT
DUMMY = N
NPAD = 10112
ROWS_PER_TILE = NPAD // NS


@functools.lru_cache(maxsize=None)
def _make_sc_agg(Dc):
    mesh = plsc.VectorSubcoreMesh(core_axis_name="c", subcore_axis_name="s")

    @functools.partial(
        pl.kernel,
        out_type=jax.ShapeDtypeStruct((NC, Dc, NPAD, 128), jnp.float32),
        mesh=mesh,
        scratch_types=[
            pltpu.VMEM((NB_E // 2, K), jnp.int32),
            pltpu.VMEM((NBUF, 2, K), jnp.int32),
            pltpu.VMEM((K, 128), jnp.float32),
            pltpu.VMEM((K, 128), jnp.float32),
            pltpu.VMEM((K, 128), jnp.float32),
            pltpu.VMEM((K, 128), jnp.float32),
            pltpu.VMEM_SHARED((NPAD, 128), jnp.float32),
            pltpu.SemaphoreType.DMA,
            pltpu.SemaphoreType.DMA,
            pltpu.SemaphoreType.DMA,
            pltpu.SemaphoreType.DMA,
        ],
    )
    def agg(h_hbm, pk_hbm, p_hbm, pk_t, sd_idx, rows0, rows1, rows2, rows3,
            agg_sh, sem0, sem1, sem2, sem3):
        cid = lax.axis_index("c")
        sid = lax.axis_index("s")
        r0 = sid * ROWS_PER_TILE
        rows = (rows0, rows1, rows2, rows3)
        sems = (sem0, sem1, sem2, sem3)

        HALF_E = NB_E // 2

        def _unpack(j, slot):
            for q in range(K // 16):
                p = pk_t[j % HALF_E, pl.ds(q * 16, 16)]
                sd_idx[slot, 0, pl.ds(q * 16, 16)] = p & 0xFFFF
                sd_idx[slot, 1, pl.ds(q * 16, 16)] = p >> 16

        for c in range(Dc):
            def _z(i, carry):
                rows0[i // 8, pl.ds((i % 8) * 16, 16)] = jnp.zeros(
                    (16,), jnp.float32)
                return carry

            lax.fori_loop(0, K * 8, _z, 0)
            for z in range(ROWS_PER_TILE // K):
                pltpu.sync_copy(rows0, agg_sh.at[pl.ds(r0 + z * K, K)])
            rem = ROWS_PER_TILE % K
            if rem:
                pltpu.sync_copy(
                    rows0.at[pl.ds(0, rem)],
                    agg_sh.at[pl.ds(r0 + ROWS_PER_TILE - rem, rem)])
            plsc.subcore_barrier()

            hc = h_hbm.at[c]

            pltpu.sync_copy(pk_hbm.at[cid, sid, pl.ds(0, HALF_E)], pk_t)

            for t in range(LA):
                _unpack(t, t)
                pltpu.async_copy(hc.at[sd_idx.at[t, 0]], rows[t], sems[t])

            def _quad(m, carry):
                for k in range(NBUF):
                    b = NBUF * m + k
                    slot = (k + LA) % NBUF

                    @pl.when(b + LA < NB_E)
                    def _():
                        @pl.when(b + LA == HALF_E)
                        def _():
                            pltpu.sync_copy(
                                pk_hbm.at[cid, sid, pl.ds(HALF_E, HALF_E)],
                                pk_t)

                        _unpack(b + LA, slot)
                        pltpu.async_copy(
                            hc.at[sd_idx.at[slot, 0]], rows[slot], sems[slot])

                    pltpu.make_async_copy(
                        hc.at[sd_idx.at[k, 0]], rows[k], sems[k]).wait()
                    pltpu.sync_copy(
                        rows[k], agg_sh.at[sd_idx.at[k, 1]], add=True)
                return carry

            lax.fori_loop(0, NB_E // NBUF, _quad, 0)
            plsc.subcore_barrier()

            pltpu.sync_copy(
                agg_sh.at[pl.ds(r0, ROWS_PER_TILE)],
                p_hbm.at[cid, c, pl.ds(r0, ROWS_PER_TILE)],
            )

    return agg


CPC = 2
NB2 = 320
SEG = 80


@functools.lru_cache(maxsize=None)
def _make_sc_agg_owned():
    mesh = plsc.VectorSubcoreMesh(core_axis_name="c", subcore_axis_name="s")

    @functools.partial(
        pl.kernel,
        out_type=jax.ShapeDtypeStruct((2 * CPC, NPAD, 128), jnp.float32),
        mesh=mesh,
        scratch_types=[
            pltpu.VMEM((SEG, K), jnp.int32),
            pltpu.VMEM((NBUF, 2, K), jnp.int32),
            pltpu.VMEM((K, 128), jnp.float32),
            pltpu.VMEM((K, 128), jnp.float32),
            pltpu.VMEM((K, 128), jnp.float32),
            pltpu.VMEM((K, 128), jnp.float32),
            pltpu.VMEM_SHARED((NPAD, 128), jnp.float32),
            pltpu.SemaphoreType.DMA,
            pltpu.SemaphoreType.DMA,
            pltpu.SemaphoreType.DMA,
            pltpu.SemaphoreType.DMA,
        ],
    )
    def agg(h_hbm, pk_hbm, p_hbm, pk_t, sd_idx, rows0, rows1, rows2, rows3,
            agg_sh, sem0, sem1, sem2, sem3):
        cid = lax.axis_index("c")
        sid = lax.axis_index("s")
        r0 = sid * ROWS_PER_TILE
        rows = (rows0, rows1, rows2, rows3)
        sems = (sem0, sem1, sem2, sem3)

        for cl in range(CPC):
            cg = cid * CPC + cl
            cbase = cg * N

            def _unpack(j, slot):
                for q in range(K // 16):
                    p = pk_t[j % SEG, pl.ds(q * 16, 16)]
                    sd_idx[slot, 0, pl.ds(q * 16, 16)] = (p & 0xFFFF) + cbase
                    sd_idx[slot, 1, pl.ds(q * 16, 16)] = p >> 16

            def _z(i, carry):
                rows0[i // 8, pl.ds((i % 8) * 16, 16)] = jnp.zeros(
                    (16,), jnp.float32)
                return carry

            lax.fori_loop(0, K * 8, _z, 0)
            for z in range(ROWS_PER_TILE // K):
                pltpu.sync_copy(rows0, agg_sh.at[pl.ds(r0 + z * K, K)])
            rem = ROWS_PER_TILE % K
            if rem:
                pltpu.sync_copy(
                    rows0.at[pl.ds(0, rem)],
                    agg_sh.at[pl.ds(r0 + ROWS_PER_TILE - rem, rem)])
            plsc.subcore_barrier()

            pltpu.sync_copy(pk_hbm.at[sid, pl.ds(0, SEG)], pk_t)
            for t in range(LA):
                _unpack(t, t)
                pltpu.async_copy(h_hbm.at[sd_idx.at[t, 0]], rows[t], sems[t])

            def _quad(m, carry):
                for k in range(NBUF):
                    b = NBUF * m + k
                    slot = (k + LA) % NBUF

                    @pl.when(b + LA < NB2)
                    def _():
                        jj = b + LA

                        @pl.when(jj % SEG == 0)
                        def _():
                            start = pl.multiple_of(jj, 16)
                            pltpu.sync_copy(
                                pk_hbm.at[sid, pl.ds(start, SEG)], pk_t)

                        _unpack(jj, slot)
                        pltpu.async_copy(
                            h_hbm.at[sd_idx.at[slot, 0]], rows[slot],
                            sems[slot])

                    pltpu.make_async_copy(
                        h_hbm.at[sd_idx.at[k, 0]], rows[k], sems[k]).wait()
                    pltpu.sync_copy(
                        rows[k], agg_sh.at[sd_idx.at[k, 1]], add=True)
                return carry

            lax.fori_loop(0, NB2 // NBUF, _quad, 0)
            plsc.subcore_barrier()

            pltpu.sync_copy(
                agg_sh.at[pl.ds(r0, ROWS_PER_TILE)],
                p_hbm.at[cg, pl.ds(r0, ROWS_PER_TILE)],
            )

    return agg


R = 1000
NBLK = N // R


def _mlp_body(Dcin, h_ref, p_ref, w1_ref, b1_ref, w2_ref, b2_ref, o_ref):
    if Dcin == 1:
        parts = [h_ref[d] + p_ref[0, d] + p_ref[1, d] for d in range(Dcin)]
    else:
        parts = [h_ref[d] + p_ref[d] for d in range(Dcin)]
    m = parts[0] if Dcin == 1 else jnp.concatenate(parts, axis=-1)
    m = jnp.dot(m, w1_ref[...], preferred_element_type=jnp.float32) + b1_ref[...]
    m = jnp.maximum(m, 0.0)
    m = jnp.dot(m, w2_ref[...], preferred_element_type=jnp.float32) + b2_ref[...]
    m = jnp.maximum(m, 0.0)
    for d in range(4):
        o_ref[d] = m[:, d * 128:(d + 1) * 128]


def _mlp(h_c, p, W1, b1, W2, b2):
    Dcin = h_c.shape[0]
    Din = Dcin * 128
    p_spec = (pl.BlockSpec((NC, Dcin, R, 128), lambda i: (0, 0, i, 0))
              if p.ndim == 4 else
              pl.BlockSpec((Dcin, R, 128), lambda i: (0, i, 0)))
    return pl.pallas_call(
        functools.partial(_mlp_body, Dcin),
        grid=(NBLK,),
        in_specs=[
            pl.BlockSpec((Dcin, R, 128), lambda i: (0, i, 0)),
            p_spec,
            pl.BlockSpec((Din, 512), lambda i: (0, 0)),
            pl.BlockSpec((1, 512), lambda i: (0, 0)),
            pl.BlockSpec((512, 512), lambda i: (0, 0)),
            pl.BlockSpec((1, 512), lambda i: (0, 0)),
        ],
        out_specs=pl.BlockSpec((4, R, 128), lambda i: (0, i, 0)),
        out_shape=jax.ShapeDtypeStruct((4, N, 128), jnp.float32),
    )(h_c, p, W1, b1.reshape(1, 512), W2, b2.reshape(1, 512))


def _pool_body(h_ref, b_ref, wl_ref, bl_ref, o_ref, g_ref, sums, cnts):
    i = pl.program_id(0)

    @pl.when(i == 0)
    def _():
        sums[...] = jnp.zeros_like(sums)
        cnts[...] = jnp.zeros_like(cnts)

    hb = jnp.concatenate([h_ref[d] for d in range(4)], axis=-1)
    b = b_ref[0]
    oh = (lax.broadcasted_iota(jnp.int32, (G, R), 0) == b).astype(jnp.float32)
    sums[...] += jnp.dot(oh, hb, preferred_element_type=jnp.float32)
    cnts[...] += jnp.broadcast_to(jnp.sum(oh, axis=1, keepdims=True), cnts.shape)

    @pl.when(i == NBLK - 1)
    def _():
        cnt = jnp.maximum(cnts[:, :1], 1.0)
        g = sums[...] / cnt
        g_ref[...] = g
        o_ref[...] = jnp.dot(g, wl_ref[...], preferred_element_type=jnp.float32) + bl_ref[...]


def _pool(h_c, batch_r, Wl, bl):
    return pl.pallas_call(
        _pool_body,
        grid=(NBLK,),
        in_specs=[
            pl.BlockSpec((4, R, 128), lambda i: (0, i, 0)),
            pl.BlockSpec((1, 1, R), lambda i: (i, 0, 0)),
            pl.BlockSpec((512, 128), lambda i: (0, 0)),
            pl.BlockSpec((1, 128), lambda i: (0, 0)),
        ],
        out_specs=[
            pl.BlockSpec((G, 128), lambda i: (0, 0)),
            pl.BlockSpec((G, 512), lambda i: (0, 0)),
        ],
        out_shape=[
            jax.ShapeDtypeStruct((G, 128), jnp.float32),
            jax.ShapeDtypeStruct((G, 512), jnp.float32),
        ],
        scratch_shapes=[
            pltpu.VMEM((G, 512), jnp.float32),
            pltpu.VMEM((G, 128), jnp.float32),
        ],
    )(h_c, batch_r, Wl, bl)


@jax.jit
def kernel(x, edge_index, batch,
           W1_0, b1_0, W2_0, b2_0,
           W1_1, b1_1, W2_1, b2_1,
           W1_2, b1_2, W2_2, b2_2,
           W_lin, b_lin):
    pad_i = lax.iota(jnp.int32, EPAD - E)
    src_p = jnp.concatenate([edge_index[0], pad_i % N])
    dst_p = jnp.concatenate([edge_index[1], DUMMY + pad_i % (NPAD - DUMMY)])
    packed = src_p | (dst_p << 16)
    pk1 = packed.reshape(NC, NS, NB_E, K)
    pk2 = packed.reshape(NS, NB2, K)

    h = x.reshape(1, N, 128)
    layer_params = [(W1_0, b1_0, W2_0, b2_0),
                    (W1_1, b1_1, W2_1, b2_1),
                    (W1_2, b1_2, W2_2, b2_2)]
    for li, (W1, b1, W2, b2) in enumerate(layer_params):
        if h.shape[0] == 1:
            p = _make_sc_agg(1)(h, pk1)
        else:
            p = _make_sc_agg_owned()(h.reshape(4 * N, 128), pk2)
        h = _mlp(h, p, W1, b1, W2, b2)

    batch_r = batch.reshape(NBLK, 1, R)
    Wl = jnp.pad(W_lin, ((0, 0), (0, 128 - C)))
    bl = jnp.pad(b_lin, (0, 128 - C)).reshape(1, 128)
    out_p, g = _pool(h, batch_r, Wl, bl)
    return (out_p[:, :C], g)
